# baseline (speedup 1.0000x reference)
import numpy as np
import jax
import jax.numpy as jnp
from jax.experimental import pallas as pl
from jax.experimental.pallas import tpu as pltpu

_D_MODEL = 4096
_N_EXPERTS = 64
_GROUP = 4
_MARGIN = 0.1
_BOOST_EPS = 0.0001
_NK = 2


def _router_kernel(x_ref, w_ref, b_ref, o_ref, acc_ref):
    k = pl.program_id(1)
    part = jax.lax.dot_general(
        x_ref[...],
        w_ref[...],
        dimension_numbers=(((1,), (1,)), ((), ())),
        preferred_element_type=jnp.float32,
    )

    @pl.when(k == 0)
    def _():
        acc_ref[...] = part + b_ref[...]

    @pl.when(k > 0)
    def _():
        acc_ref[...] = acc_ref[...] + part

    @pl.when(k == _NK - 1)
    def _():
        logits = acc_ref[...]
        bt = logits.shape[0]
        lane = jax.lax.broadcasted_iota(jnp.int32, (bt, _N_EXPERTS), 1)
        even = (lane & 1) == 0
        low2 = (lane & 2) == 0

        y = jnp.maximum(
            logits,
            jnp.where(even, pltpu.roll(logits, 63, 1), pltpu.roll(logits, 1, 1)),
        )
        mx = jnp.maximum(
            y, jnp.where(low2, pltpu.roll(y, 62, 1), pltpu.roll(y, 2, 1))
        )

        w = ((mx - logits) < _MARGIN).astype(jnp.float32)
        c = w + jnp.where(even, pltpu.roll(w, 63, 1), pltpu.roll(w, 1, 1))
        cnt = c + jnp.where(low2, pltpu.roll(c, 62, 1), pltpu.roll(c, 2, 1))

        overwrite = ((lane & (_GROUP - 1)) == 0) & (cnt > 1.5)
        o_ref[...] = jnp.where(overwrite, mx + _BOOST_EPS, logits)


def kernel(hidden_states, W, b):
    T, D = hidden_states.shape
    BT = 2048
    BK = D // _NK
    b2 = b.reshape(1, _N_EXPERTS)
    return pl.pallas_call(
        _router_kernel,
        grid=(T // BT, _NK),
        in_specs=[
            pl.BlockSpec((BT, BK), lambda i, k: (i, k)),
            pl.BlockSpec((_N_EXPERTS, BK), lambda i, k: (0, k)),
            pl.BlockSpec((1, _N_EXPERTS), lambda i, k: (0, 0)),
        ],
        out_specs=pl.BlockSpec((BT, _N_EXPERTS), lambda i, k: (i, 0)),
        out_shape=jax.ShapeDtypeStruct((T, _N_EXPERTS), jnp.float32),
        scratch_shapes=[pltpu.VMEM((BT, _N_EXPERTS), jnp.float32)],
        compiler_params=pltpu.CompilerParams(
            dimension_semantics=("parallel", "arbitrary"),
        ),
    )(hidden_states, W, b2)


# speedup vs baseline: 1.0025x; 1.0025x over previous
import numpy as np
import jax
import jax.numpy as jnp
from jax.experimental import pallas as pl
from jax.experimental.pallas import tpu as pltpu

_D_MODEL = 4096
_N_EXPERTS = 64
_GROUP = 4
_MARGIN = 0.1
_BOOST_EPS = 0.0001
_CT = 1024
_NCHUNK = 16
_NBUF = 3
_NOBUF = 2


def _canonicalize(logits):
    bt = logits.shape[0]
    lane = jax.lax.broadcasted_iota(jnp.int32, (bt, _N_EXPERTS), 1)
    even = (lane & 1) == 0
    low2 = (lane & 2) == 0

    y = jnp.maximum(
        logits,
        jnp.where(even, pltpu.roll(logits, 63, 1), pltpu.roll(logits, 1, 1)),
    )
    mx = jnp.maximum(
        y, jnp.where(low2, pltpu.roll(y, 62, 1), pltpu.roll(y, 2, 1))
    )

    w = ((mx - logits) < _MARGIN).astype(jnp.float32)
    c = w + jnp.where(even, pltpu.roll(w, 63, 1), pltpu.roll(w, 1, 1))
    cnt = c + jnp.where(low2, pltpu.roll(c, 62, 1), pltpu.roll(c, 2, 1))

    overwrite = ((lane & (_GROUP - 1)) == 0) & (cnt > 1.5)
    return jnp.where(overwrite, mx + _BOOST_EPS, logits)


def _router_kernel(x_hbm, w_ref, b_ref, o_hbm, xbuf, obuf, in_sem, out_sem):
    def in_copy(c, slot):
        return pltpu.make_async_copy(
            x_hbm.at[pl.ds(c * _CT, _CT), :], xbuf.at[slot], in_sem.at[slot]
        )

    def out_copy(c, slot):
        return pltpu.make_async_copy(
            obuf.at[slot], o_hbm.at[pl.ds(c * _CT, _CT), :], out_sem.at[slot]
        )

    for s in range(_NBUF):
        in_copy(s, s).start()

    w = w_ref[...]
    b = b_ref[...]
    for c in range(_NCHUNK):
        slot = c % _NBUF
        in_copy(c, slot).wait()
        logits = jax.lax.dot_general(
            xbuf[slot],
            w,
            dimension_numbers=(((1,), (1,)), ((), ())),
            preferred_element_type=jnp.float32,
        )
        out = _canonicalize(logits + b)
        oslot = c % _NOBUF
        if c >= _NOBUF:
            out_copy(c - _NOBUF, oslot).wait()
        obuf[oslot] = out
        out_copy(c, oslot).start()
        nxt = c + _NBUF
        if nxt < _NCHUNK:
            in_copy(nxt, slot).start()
    for c in range(_NCHUNK - _NOBUF, _NCHUNK):
        out_copy(c, c % _NOBUF).wait()


def kernel(hidden_states, W, b):
    T, D = hidden_states.shape
    b2 = b.reshape(1, _N_EXPERTS)
    return pl.pallas_call(
        _router_kernel,
        in_specs=[
            pl.BlockSpec(memory_space=pltpu.MemorySpace.HBM),
            pl.BlockSpec(memory_space=pltpu.MemorySpace.VMEM),
            pl.BlockSpec(memory_space=pltpu.MemorySpace.VMEM),
        ],
        out_specs=pl.BlockSpec(memory_space=pltpu.MemorySpace.HBM),
        out_shape=jax.ShapeDtypeStruct((T, _N_EXPERTS), jnp.float32),
        scratch_shapes=[
            pltpu.VMEM((_NBUF, _CT, _D_MODEL), jnp.float32),
            pltpu.VMEM((_NOBUF, _CT, _N_EXPERTS), jnp.float32),
            pltpu.SemaphoreType.DMA((_NBUF,)),
            pltpu.SemaphoreType.DMA((_NOBUF,)),
        ],
    )(hidden_states, W, b2)


# speedup vs baseline: 1.1920x; 1.1891x over previous
import numpy as np
import jax
import jax.numpy as jnp
from jax.experimental import pallas as pl
from jax.experimental.pallas import tpu as pltpu

_D_MODEL = 4096
_N_EXPERTS = 64
_GROUP = 4
_MARGIN = 0.1
_BOOST_EPS = 0.0001
_CT = 1024
_NCHUNK = 16
_NBUF = 3
_NOBUF = 2


def _canonicalize(logits):
    bt = logits.shape[0]
    lane = jax.lax.broadcasted_iota(jnp.int32, (bt, _N_EXPERTS), 1)
    even = (lane & 1) == 0
    low2 = (lane & 2) == 0

    y = jnp.maximum(
        logits,
        jnp.where(even, pltpu.roll(logits, 63, 1), pltpu.roll(logits, 1, 1)),
    )
    mx = jnp.maximum(
        y, jnp.where(low2, pltpu.roll(y, 62, 1), pltpu.roll(y, 2, 1))
    )

    w = ((mx - logits) < _MARGIN).astype(jnp.float32)
    c = w + jnp.where(even, pltpu.roll(w, 63, 1), pltpu.roll(w, 1, 1))
    cnt = c + jnp.where(low2, pltpu.roll(c, 62, 1), pltpu.roll(c, 2, 1))

    overwrite = ((lane & (_GROUP - 1)) == 0) & (cnt > 1.5)
    return jnp.where(overwrite, mx + _BOOST_EPS, logits)


def _router_kernel(x_hbm, w_ref, b_ref, o_hbm, xbuf, obuf, in_sem, out_sem):
    def in_copy(c, slot):
        return pltpu.make_async_copy(
            x_hbm.at[pl.ds(c * _CT, _CT), :], xbuf.at[slot], in_sem.at[slot]
        )

    def out_copy(c, slot):
        return pltpu.make_async_copy(
            obuf.at[slot], o_hbm.at[pl.ds(c * _CT, _CT), :], out_sem.at[slot]
        )

    for s in range(_NBUF):
        in_copy(s, s).start()

    def body(c, carry):
        slot = jax.lax.rem(c, _NBUF)
        oslot = jax.lax.rem(c, _NOBUF)
        in_copy(c, slot).wait()
        logits = jax.lax.dot_general(
            xbuf[slot],
            w_ref[...],
            dimension_numbers=(((1,), (1,)), ((), ())),
            preferred_element_type=jnp.float32,
        )
        out = _canonicalize(logits + b_ref[...])

        @pl.when(c >= _NOBUF)
        def _():
            out_copy(c - _NOBUF, oslot).wait()

        obuf[oslot] = out
        out_copy(c, oslot).start()

        @pl.when(c + _NBUF < _NCHUNK)
        def _():
            in_copy(c + _NBUF, slot).start()

        return carry

    jax.lax.fori_loop(0, _NCHUNK, body, 0)
    for c in range(_NCHUNK - _NOBUF, _NCHUNK):
        out_copy(c, c % _NOBUF).wait()


def kernel(hidden_states, W, b):
    T, D = hidden_states.shape
    b2 = b.reshape(1, _N_EXPERTS)
    return pl.pallas_call(
        _router_kernel,
        in_specs=[
            pl.BlockSpec(memory_space=pltpu.MemorySpace.HBM),
            pl.BlockSpec(memory_space=pltpu.MemorySpace.VMEM),
            pl.BlockSpec(memory_space=pltpu.MemorySpace.VMEM),
        ],
        out_specs=pl.BlockSpec(memory_space=pltpu.MemorySpace.HBM),
        out_shape=jax.ShapeDtypeStruct((T, _N_EXPERTS), jnp.float32),
        scratch_shapes=[
            pltpu.VMEM((_NBUF, _CT, _D_MODEL), jnp.float32),
            pltpu.VMEM((_NOBUF, _CT, _N_EXPERTS), jnp.float32),
            pltpu.SemaphoreType.DMA((_NBUF,)),
            pltpu.SemaphoreType.DMA((_NOBUF,)),
        ],
    )(hidden_states, W, b2)


# speedup vs baseline: 1.1945x; 1.0020x over previous
import numpy as np
import jax
import jax.numpy as jnp
from jax.experimental import pallas as pl
from jax.experimental.pallas import tpu as pltpu

_D_MODEL = 4096
_N_EXPERTS = 64
_GROUP = 4
_MARGIN = 0.1
_BOOST_EPS = 0.0001
_CT = 1024
_NCHUNK = 16
_NBUF = 3
_NOBUF = 2


def _canonicalize(logits):
    bt = logits.shape[0]
    lane = jax.lax.broadcasted_iota(jnp.int32, (bt, _N_EXPERTS), 1)
    even = (lane & 1) == 0
    low2 = (lane & 2) == 0

    y = jnp.maximum(
        logits,
        jnp.where(even, pltpu.roll(logits, 63, 1), pltpu.roll(logits, 1, 1)),
    )
    mx = jnp.maximum(
        y, jnp.where(low2, pltpu.roll(y, 62, 1), pltpu.roll(y, 2, 1))
    )

    w = ((mx - logits) < _MARGIN).astype(jnp.float32)
    c = w + jnp.where(even, pltpu.roll(w, 63, 1), pltpu.roll(w, 1, 1))
    cnt = c + jnp.where(low2, pltpu.roll(c, 62, 1), pltpu.roll(c, 2, 1))

    overwrite = ((lane & (_GROUP - 1)) == 0) & (cnt > 1.5)
    return jnp.where(overwrite, mx + _BOOST_EPS, logits)


def _router_kernel(x_hbm, w_ref, b_ref, o_hbm, xbuf, obuf, in_sem, out_sem):
    half = _CT // 2

    def in_copies(c, slot):
        lo = pltpu.make_async_copy(
            x_hbm.at[pl.ds(c * _CT, half), :],
            xbuf.at[slot, pl.ds(0, half), :],
            in_sem.at[slot, 0],
        )
        hi = pltpu.make_async_copy(
            x_hbm.at[pl.ds(c * _CT + half, half), :],
            xbuf.at[slot, pl.ds(half, half), :],
            in_sem.at[slot, 1],
        )
        return lo, hi

    def out_copy(c, slot):
        return pltpu.make_async_copy(
            obuf.at[slot], o_hbm.at[pl.ds(c * _CT, _CT), :], out_sem.at[slot]
        )

    for s in range(_NBUF):
        for cp in in_copies(s, s):
            cp.start()

    def body(c, carry):
        slot = jax.lax.rem(c, _NBUF)
        oslot = jax.lax.rem(c, _NOBUF)
        for cp in in_copies(c, slot):
            cp.wait()
        logits = jax.lax.dot_general(
            xbuf[slot],
            w_ref[...],
            dimension_numbers=(((1,), (1,)), ((), ())),
            preferred_element_type=jnp.float32,
        )
        out = _canonicalize(logits + b_ref[...])

        @pl.when(c >= _NOBUF)
        def _():
            out_copy(c - _NOBUF, oslot).wait()

        obuf[oslot] = out
        out_copy(c, oslot).start()

        @pl.when(c + _NBUF < _NCHUNK)
        def _():
            for cp in in_copies(c + _NBUF, slot):
                cp.start()

        return carry

    jax.lax.fori_loop(0, _NCHUNK, body, 0)
    for c in range(_NCHUNK - _NOBUF, _NCHUNK):
        out_copy(c, c % _NOBUF).wait()


def kernel(hidden_states, W, b):
    T, D = hidden_states.shape
    b2 = b.reshape(1, _N_EXPERTS)
    return pl.pallas_call(
        _router_kernel,
        in_specs=[
            pl.BlockSpec(memory_space=pltpu.MemorySpace.HBM),
            pl.BlockSpec(memory_space=pltpu.MemorySpace.VMEM),
            pl.BlockSpec(memory_space=pltpu.MemorySpace.VMEM),
        ],
        out_specs=pl.BlockSpec(memory_space=pltpu.MemorySpace.HBM),
        out_shape=jax.ShapeDtypeStruct((T, _N_EXPERTS), jnp.float32),
        scratch_shapes=[
            pltpu.VMEM((_NBUF, _CT, _D_MODEL), jnp.float32),
            pltpu.VMEM((_NOBUF, _CT, _N_EXPERTS), jnp.float32),
            pltpu.SemaphoreType.DMA((_NBUF, 2)),
            pltpu.SemaphoreType.DMA((_NOBUF,)),
        ],
    )(hidden_states, W, b2)


# speedup vs baseline: 1.2255x; 1.0260x over previous
import numpy as np
import jax
import jax.numpy as jnp
from jax.experimental import pallas as pl
from jax.experimental.pallas import tpu as pltpu

_D_MODEL = 4096
_N_EXPERTS = 64
_GROUP = 4
_MARGIN = 0.1
_BOOST_EPS = 0.0001


def _router_kernel(x_ref, w_ref, b_ref, o_ref):
    x = x_ref[...]
    logits = jax.lax.dot_general(
        x,
        w_ref[...],
        dimension_numbers=(((1,), (1,)), ((), ())),
        preferred_element_type=jnp.float32,
    )
    logits = logits + b_ref[...]

    bt = logits.shape[0]
    lane = jax.lax.broadcasted_iota(jnp.int32, (bt, _N_EXPERTS), 1)
    even = (lane & 1) == 0
    low2 = (lane & 2) == 0

    y = jnp.maximum(
        logits,
        jnp.where(even, pltpu.roll(logits, 63, 1), pltpu.roll(logits, 1, 1)),
    )
    mx = jnp.maximum(
        y, jnp.where(low2, pltpu.roll(y, 62, 1), pltpu.roll(y, 2, 1))
    )

    w = ((mx - logits) < _MARGIN).astype(jnp.float32)
    c = w + jnp.where(even, pltpu.roll(w, 63, 1), pltpu.roll(w, 1, 1))
    cnt = c + jnp.where(low2, pltpu.roll(c, 62, 1), pltpu.roll(c, 2, 1))

    overwrite = ((lane & (_GROUP - 1)) == 0) & (cnt > 1.5)
    o_ref[...] = jnp.where(overwrite, mx + _BOOST_EPS, logits)


def kernel(hidden_states, W, b):
    T, D = hidden_states.shape
    BT = 1024
    b2 = b.reshape(1, _N_EXPERTS)
    return pl.pallas_call(
        _router_kernel,
        grid=(T // BT,),
        in_specs=[
            pl.BlockSpec((BT, D), lambda i: (i, 0)),
            pl.BlockSpec((_N_EXPERTS, D), lambda i: (0, 0)),
            pl.BlockSpec((1, _N_EXPERTS), lambda i: (0, 0)),
        ],
        out_specs=pl.BlockSpec((BT, _N_EXPERTS), lambda i: (i, 0)),
        out_shape=jax.ShapeDtypeStruct((T, _N_EXPERTS), jnp.float32),
        compiler_params=pltpu.CompilerParams(
            dimension_semantics=("arbitrary",),
        ),
    )(hidden_states, W, b2)
